# baseline (speedup 1.0000x reference)
import jax
import jax.numpy as jnp
from jax.experimental import pallas as pl


def _relu_body(x_ref, o_ref):
    o_ref[...] = jnp.maximum(x_ref[...], 0.0)


def kernel(X, v_ids, e_ids, e_weight, W, b):
    N, E = X.shape[0], e_weight.shape[0]
    H = X @ W.T + b
    ones = jnp.ones((v_ids.shape[0],), dtype=jnp.float32)
    de = jnp.clip(jax.ops.segment_sum(ones, e_ids, num_segments=E), 1.0, None)
    Y = jax.ops.segment_sum(jnp.take(H, v_ids, axis=0), e_ids, num_segments=E) / de[:, None]
    Y = e_weight[:, None] * Y
    dv = jnp.clip(jax.ops.segment_sum(ones, v_ids, num_segments=N), 1.0, None)
    out = jax.ops.segment_sum(jnp.take(Y, e_ids, axis=0), v_ids, num_segments=N) / dv[:, None]
    return pl.pallas_call(
        _relu_body,
        out_shape=jax.ShapeDtypeStruct(out.shape, out.dtype),
    )(out)


# speedup vs baseline: 3.7437x; 3.7437x over previous
import functools

import jax
import jax.numpy as jnp
from jax import lax
from jax.experimental import pallas as pl
from jax.experimental.pallas import tpu as pltpu
from jax.experimental.pallas import tpu_sc as plsc

N = 10000
E = 2000
NNZ = 100000
D = 512

N_PAD = 10240
NUM_TILES = 16
PPT = 6272
NNZ_PAD = NUM_TILES * PPT
IDX_ROWS = PPT // 128

CHUNK_E = 125
NCHUNK = E // CHUNK_E
CHUNK_EL = CHUNK_E * N_PAD
DUMMY = CHUNK_EL
SPMEM_EL = CHUNK_EL + 64
TILE_EL = CHUNK_EL // NUM_TILES
ZW = TILE_EL // 8


def _build_b_body(v_hbm, e_hbm, b_hbm, v_v, e_v, idx_v, ones_v, zeros_v, stage_v,
                  acc_sh):
    cid = lax.axis_index("c")
    sid = lax.axis_index("s")

    ones16 = jnp.ones((16,), jnp.int32)
    zeros16 = jnp.zeros((16,), jnp.int32)
    for g in range(8):
        ones_v[pl.ds(g * 16, 16)] = ones16

    def _zinit(i, carry):
        zeros_v[pl.ds(i * 16, 16)] = zeros16
        return carry
    lax.fori_loop(0, ZW // 16, _zinit, 0)

    base = sid * PPT
    pltpu.sync_copy(v_hbm.at[pl.ds(base, PPT)], v_v)
    pltpu.sync_copy(e_hbm.at[pl.ds(base, PPT)], e_v)

    def one_chunk(p, carry):
        c = 2 * p + cid
        lo = c * CHUNK_E

        for z in range(8):
            pltpu.sync_copy(zeros_v, acc_sh.at[pl.ds(sid * TILE_EL + z * ZW, ZW)])

        def idx_row(r, carry2):
            for g in range(8):
                off = r * 128 + g * 16
                e16 = e_v[pl.ds(off, 16)]
                v16 = v_v[pl.ds(off, 16)]
                rel = e16 - lo
                inr = (rel >= 0) & (rel < CHUNK_E)
                flat = rel * N_PAD + v16
                idx_v[r, pl.ds(g * 16, 16)] = jnp.where(inr, flat, DUMMY)
            return carry2
        lax.fori_loop(0, IDX_ROWS, idx_row, 0)

        plsc.subcore_barrier()

        def scat(r, carry2):
            pltpu.sync_copy(ones_v, acc_sh.at[idx_v.at[r]], add=True)
            return carry2
        lax.fori_loop(0, IDX_ROWS, scat, 0)

        plsc.subcore_barrier()
        pltpu.sync_copy(acc_sh.at[pl.ds(0, 16)], stage_v.at[pl.ds(0, 16)])
        plsc.subcore_barrier()

        for z in range(8):
            off = sid * TILE_EL + z * ZW
            pltpu.sync_copy(acc_sh.at[pl.ds(off, ZW)], stage_v)
            pltpu.sync_copy(stage_v, b_hbm.at[pl.ds(c * CHUNK_EL + off, ZW)])
        return carry

    lax.fori_loop(0, NCHUNK // 2, one_chunk, 0)


@functools.cache
def _build_b():
    return pl.kernel(
        _build_b_body,
        out_type=jax.ShapeDtypeStruct((E * N_PAD,), jnp.int32),
        mesh=plsc.VectorSubcoreMesh(core_axis_name="c", subcore_axis_name="s",
                                    num_cores=2, num_subcores=NUM_TILES),
        scratch_types=[
            pltpu.VMEM((PPT,), jnp.int32),
            pltpu.VMEM((PPT,), jnp.int32),
            pltpu.VMEM((IDX_ROWS, 128), jnp.int32),
            pltpu.VMEM((128,), jnp.int32),
            pltpu.VMEM((ZW,), jnp.int32),
            pltpu.VMEM((ZW,), jnp.int32),
            pltpu.VMEM_SHARED((SPMEM_EL,), jnp.int32),
        ],
    )


BK = 512
NK = N_PAD // BK
DX = D + 128


def _v2e_body(b_ref, x_ref, w_ref, ew_ref, bias_ref, out_ref, acc):
    j = pl.program_id(0)

    @pl.when(j == 0)
    def _():
        acc[...] = jnp.zeros_like(acc)

    acc[...] += jnp.dot(b_ref[...].astype(jnp.bfloat16), x_ref[...],
                        preferred_element_type=jnp.float32)

    @pl.when(j == NK - 1)
    def _():
        a = acc[...]
        de = jnp.sum(a[:, D:], axis=1, keepdims=True) * (1.0 / 128.0)
        w = ew_ref[...]
        scale = w / jnp.maximum(de, 1.0)
        xm = (a[:, :D] * scale).astype(jnp.bfloat16)
        h = lax.dot_general(xm, w_ref[...].astype(jnp.bfloat16),
                            (((1,), (1,)), ((), ())),
                            preferred_element_type=jnp.float32)
        y = (h + w * bias_ref[...]).astype(jnp.bfloat16)
        out_ref[...] = jnp.concatenate(
            [y, jnp.full((E, 128), 1.0, jnp.bfloat16)], axis=1)


def _v2e(Bm, Xc, W, ew, bias):
    return pl.pallas_call(
        _v2e_body,
        grid=(NK,),
        in_specs=[
            pl.BlockSpec((E, BK), lambda j: (0, j)),
            pl.BlockSpec((BK, DX), lambda j: (j, 0)),
            pl.BlockSpec((D, D), lambda j: (0, 0)),
            pl.BlockSpec((E, 1), lambda j: (0, 0)),
            pl.BlockSpec((1, D), lambda j: (0, 0)),
        ],
        out_specs=pl.BlockSpec((E, DX), lambda j: (0, 0)),
        out_shape=jax.ShapeDtypeStruct((E, DX), jnp.bfloat16),
        scratch_shapes=[
            pltpu.VMEM((E, DX), jnp.float32),
        ],
    )(Bm, Xc, W, ew, bias)


BN = 1024
NBLK = N_PAD // BN


def _e2v_body(b_ref, y_ref, out_ref):
    o = lax.dot_general(b_ref[...].astype(jnp.bfloat16), y_ref[...],
                        (((0,), (0,)), ((), ())),
                        preferred_element_type=jnp.float32)
    dv = jnp.sum(o[:, D:], axis=1, keepdims=True) * (1.0 / 128.0)
    out_ref[...] = jnp.maximum(o[:, :D], 0.0) / jnp.maximum(dv, 1.0)


def _e2v(Bm, Y2):
    return pl.pallas_call(
        _e2v_body,
        grid=(NBLK,),
        in_specs=[
            pl.BlockSpec((E, BN), lambda i: (0, i)),
            pl.BlockSpec((E, DX), lambda i: (0, 0)),
        ],
        out_specs=pl.BlockSpec((BN, D), lambda i: (i, 0)),
        out_shape=jax.ShapeDtypeStruct((N, D), jnp.float32),
    )(Bm, Y2)


def kernel(X, v_ids, e_ids, e_weight, W, b):
    pad = NNZ_PAD - NNZ
    v_p = jnp.pad(v_ids, (0, pad))
    e_p = jnp.pad(e_ids, (0, pad), constant_values=-1)
    Bm = _build_b()(v_p, e_p).reshape(E, N_PAD)
    Xc = jnp.concatenate(
        [X.astype(jnp.bfloat16), jnp.ones((N, 128), jnp.bfloat16)], axis=1)
    Xc = jnp.pad(Xc, ((0, N_PAD - N), (0, 0)))
    Y2 = _v2e(Bm, Xc, W, e_weight.reshape(E, 1), b.reshape(1, D))
    return _e2v(Bm, Y2)


# speedup vs baseline: 3.8229x; 1.0212x over previous
import functools

import jax
import jax.numpy as jnp
from jax import lax
from jax.experimental import pallas as pl
from jax.experimental.pallas import tpu as pltpu
from jax.experimental.pallas import tpu_sc as plsc

N = 10000
E = 2000
NNZ = 100000
D = 512

N_PAD = 10240
NUM_TILES = 16
PPT = 6272
NNZ_PAD = NUM_TILES * PPT
IDX_ROWS = PPT // 128

CHUNK_E = 125
NCHUNK = E // CHUNK_E
CHUNK_EL = CHUNK_E * N_PAD
DUMMY = CHUNK_EL
SPMEM_EL = CHUNK_EL + 64
TILE_EL = CHUNK_EL // NUM_TILES
ZW = TILE_EL // 8
SW = TILE_EL // 4


def _build_b_body(v_hbm, e_hbm, b_hbm, v_v, e_v, idx_v, ones_v, zeros_v, stage_v,
                  acc_sh, zsem, ssem, osem):
    cid = lax.axis_index("c")
    sid = lax.axis_index("s")

    ones16 = jnp.ones((16,), jnp.int32)
    zeros16 = jnp.zeros((16,), jnp.int32)
    for g in range(8):
        ones_v[pl.ds(g * 16, 16)] = ones16

    def _zinit(i, carry):
        zeros_v[pl.ds(i * 16, 16)] = zeros16
        return carry
    lax.fori_loop(0, ZW // 16, _zinit, 0)

    base = sid * PPT
    pltpu.sync_copy(v_hbm.at[pl.ds(base, PPT)], v_v)
    pltpu.sync_copy(e_hbm.at[pl.ds(base, PPT)], e_v)

    def one_chunk(p, carry):
        c = 2 * p + cid
        lo = c * CHUNK_E

        zd = [pltpu.async_copy(
                  zeros_v, acc_sh.at[pl.ds(sid * TILE_EL + z * ZW, ZW)], zsem)
              for z in range(8)]

        def idx_row(r, carry2):
            for g in range(8):
                off = r * 128 + g * 16
                e16 = e_v[pl.ds(off, 16)]
                v16 = v_v[pl.ds(off, 16)]
                rel = e16 - lo
                inr = (rel >= 0) & (rel < CHUNK_E)
                flat = rel * N_PAD + v16
                idx_v[r, pl.ds(g * 16, 16)] = jnp.where(inr, flat, DUMMY)
            return carry2
        lax.fori_loop(0, IDX_ROWS, idx_row, 0)

        for d in zd:
            d.wait()
        plsc.subcore_barrier()

        sd = [pltpu.async_copy(ones_v, acc_sh.at[idx_v.at[r]], ssem, add=True)
              for r in range(IDX_ROWS)]
        for d in sd:
            d.wait()

        plsc.subcore_barrier()
        pltpu.sync_copy(acc_sh.at[pl.ds(0, 16)], stage_v.at[pl.ds(0, 16)])
        plsc.subcore_barrier()

        for z in range(4):
            if z == 0:
                @pl.when(p > 0)
                def _():
                    pltpu.make_async_copy(
                        stage_v, b_hbm.at[pl.ds(0, SW)], osem).wait()
            else:
                pltpu.make_async_copy(
                    stage_v, b_hbm.at[pl.ds(0, SW)], osem).wait()
            off = sid * TILE_EL + z * SW
            pltpu.sync_copy(acc_sh.at[pl.ds(off, SW)], stage_v)
            pltpu.async_copy(stage_v,
                             b_hbm.at[pl.ds(c * CHUNK_EL + off, SW)], osem)
        return carry

    lax.fori_loop(0, NCHUNK // 2, one_chunk, 0)
    pltpu.make_async_copy(stage_v, b_hbm.at[pl.ds(0, SW)], osem).wait()


@functools.cache
def _build_b():
    return pl.kernel(
        _build_b_body,
        out_type=jax.ShapeDtypeStruct((E * N_PAD,), jnp.int32),
        mesh=plsc.VectorSubcoreMesh(core_axis_name="c", subcore_axis_name="s",
                                    num_cores=2, num_subcores=NUM_TILES),
        scratch_types=[
            pltpu.VMEM((PPT,), jnp.int32),
            pltpu.VMEM((PPT,), jnp.int32),
            pltpu.VMEM((IDX_ROWS, 128), jnp.int32),
            pltpu.VMEM((128,), jnp.int32),
            pltpu.VMEM((ZW,), jnp.int32),
            pltpu.VMEM((SW,), jnp.int32),
            pltpu.VMEM_SHARED((SPMEM_EL,), jnp.int32),
            pltpu.SemaphoreType.DMA,
            pltpu.SemaphoreType.DMA,
            pltpu.SemaphoreType.DMA,
        ],
    )


BK = 512
NK = N_PAD // BK
DX = D + 128


def _v2e_body(b_ref, x_ref, w_ref, ew_ref, bias_ref, out_ref, acc):
    j = pl.program_id(0)

    @pl.when(j == 0)
    def _():
        acc[...] = jnp.zeros_like(acc)

    acc[...] += jnp.dot(b_ref[...].astype(jnp.bfloat16), x_ref[...],
                        preferred_element_type=jnp.float32)

    @pl.when(j == NK - 1)
    def _():
        a = acc[...]
        de = jnp.sum(a[:, D:], axis=1, keepdims=True) * (1.0 / 128.0)
        w = ew_ref[...]
        scale = w / jnp.maximum(de, 1.0)
        xm = (a[:, :D] * scale).astype(jnp.bfloat16)
        h = lax.dot_general(xm, w_ref[...].astype(jnp.bfloat16),
                            (((1,), (1,)), ((), ())),
                            preferred_element_type=jnp.float32)
        y = (h + w * bias_ref[...]).astype(jnp.bfloat16)
        out_ref[...] = jnp.concatenate(
            [y, jnp.full((E, 128), 1.0, jnp.bfloat16)], axis=1)


def _v2e(Bm, Xc, W, ew, bias):
    return pl.pallas_call(
        _v2e_body,
        grid=(NK,),
        in_specs=[
            pl.BlockSpec((E, BK), lambda j: (0, j)),
            pl.BlockSpec((BK, DX), lambda j: (j, 0)),
            pl.BlockSpec((D, D), lambda j: (0, 0)),
            pl.BlockSpec((E, 1), lambda j: (0, 0)),
            pl.BlockSpec((1, D), lambda j: (0, 0)),
        ],
        out_specs=pl.BlockSpec((E, DX), lambda j: (0, 0)),
        out_shape=jax.ShapeDtypeStruct((E, DX), jnp.bfloat16),
        scratch_shapes=[
            pltpu.VMEM((E, DX), jnp.float32),
        ],
    )(Bm, Xc, W, ew, bias)


BN = 1024
NBLK = N_PAD // BN


def _e2v_body(b_ref, y_ref, out_ref):
    o = lax.dot_general(b_ref[...].astype(jnp.bfloat16), y_ref[...],
                        (((0,), (0,)), ((), ())),
                        preferred_element_type=jnp.float32)
    dv = jnp.sum(o[:, D:], axis=1, keepdims=True) * (1.0 / 128.0)
    out_ref[...] = jnp.maximum(o[:, :D], 0.0) / jnp.maximum(dv, 1.0)


def _e2v(Bm, Y2):
    return pl.pallas_call(
        _e2v_body,
        grid=(NBLK,),
        in_specs=[
            pl.BlockSpec((E, BN), lambda i: (0, i)),
            pl.BlockSpec((E, DX), lambda i: (0, 0)),
        ],
        out_specs=pl.BlockSpec((BN, D), lambda i: (i, 0)),
        out_shape=jax.ShapeDtypeStruct((N, D), jnp.float32),
    )(Bm, Y2)


def kernel(X, v_ids, e_ids, e_weight, W, b):
    pad = NNZ_PAD - NNZ
    v_p = jnp.pad(v_ids, (0, pad))
    e_p = jnp.pad(e_ids, (0, pad), constant_values=-1)
    Bm = _build_b()(v_p, e_p).reshape(E, N_PAD)
    Xc = jnp.concatenate(
        [X.astype(jnp.bfloat16), jnp.ones((N, 128), jnp.bfloat16)], axis=1)
    Xc = jnp.pad(Xc, ((0, N_PAD - N), (0, 0)))
    Y2 = _v2e(Bm, Xc, W, e_weight.reshape(E, 1), b.reshape(1, D))
    return _e2v(Bm, Y2)
